# Initial kernel scaffold; baseline (speedup 1.0000x reference)
import jax
import jax.numpy as jnp
from jax.experimental import pallas as pl


def kernel(inp, label):
    raise NotImplementedError("write your pallas kernel here")


# speedup vs baseline: 2.3170x; 2.3170x over previous
import functools

import jax
import jax.numpy as jnp
from jax import lax
from jax.experimental import pallas as pl
from jax.experimental.pallas import tpu as pltpu
from jax.experimental.pallas import tpu_sc as plsc

_M = 0.25
_GAMMA = 64.0
_B = 1024
_V = 100000
_CB = 2560
_NCB = -(-_V // _CB)
_NEG = -1e30


_NW = 32
_BPW = _B // _NW


def _sc_gather_body(tab_hbm, lab_hbm, out_hbm, lab_v, idx_v, g_v, sem):
    c = lax.axis_index("c")
    s = lax.axis_index("s")
    wid = s * 2 + c
    base = wid * _BPW
    pltpu.sync_copy(lab_hbm.at[pl.ds(base, _BPW)], lab_v)
    for j in range(_BPW // 16):
        sl = pl.ds(j * 16, 16)
        bvec = lax.iota(jnp.int32, 16) + (base + j * 16)
        idx_v[sl] = bvec * _V + lab_v[sl]
    pltpu.async_copy(tab_hbm.at[idx_v], g_v, sem).wait()
    pltpu.sync_copy(g_v, out_hbm.at[pl.ds(base, _BPW)])


@functools.lru_cache(maxsize=1)
def _sc_gather():
    return pl.kernel(
        _sc_gather_body,
        out_type=jax.ShapeDtypeStruct((_B,), jnp.float32),
        mesh=plsc.VectorSubcoreMesh(core_axis_name="c", subcore_axis_name="s"),
        scratch_types=[
            pltpu.VMEM((_BPW,), jnp.int32),
            pltpu.VMEM((_BPW,), jnp.int32),
            pltpu.VMEM((_BPW,), jnp.float32),
            pltpu.SemaphoreType.DMA,
        ],
    )


def _tc_body(lab_ref, g_ref, x_ref, out_ref, m_scr, s_scr):
    cb = pl.program_id(0)

    @pl.when(cb == 0)
    def _init():
        m_scr[...] = jnp.full((_B, 1), _NEG, dtype=jnp.float32)
        s_scr[...] = jnp.zeros((_B, 1), dtype=jnp.float32)

    x = x_ref[...]
    l0 = jnp.where(x > -_M, _GAMMA * (x * x) - (_GAMMA * _M * _M), 0.0)
    lab_loc = lab_ref[...] - cb * _CB
    col = lax.broadcasted_iota(jnp.int32, (_B, _CB), 1)
    bad = (col == lab_loc) | (col >= _V - cb * _CB)
    logit = jnp.where(bad, _NEG, l0)

    bm = jnp.max(logit, axis=1, keepdims=True)
    m_old = m_scr[...]
    m_new = jnp.maximum(m_old, bm)
    p = jnp.exp(logit - m_new)
    s_scr[...] = s_scr[...] * jnp.exp(m_old - m_new) + jnp.sum(
        p, axis=1, keepdims=True
    )
    m_scr[...] = m_new

    @pl.when(cb == _NCB - 1)
    def _finish():
        g = g_ref[...]
        l_c = _GAMMA * jnp.maximum(1.0 + _M - g, 0.0) * (g - (1.0 - _M))
        m_w = m_scr[...]
        mx = jnp.maximum(m_w, l_c)
        sm = s_scr[...] * jnp.exp(m_w - mx) + jnp.exp(l_c - mx)
        nll = mx + jnp.log(sm) - l_c
        out_ref[0, 0] = jnp.sum(nll) * (1.0 / _B)


_tc_loss = pl.pallas_call(
    _tc_body,
    grid=(_NCB,),
    in_specs=[
        pl.BlockSpec((_B, 1), lambda cb: (0, 0)),
        pl.BlockSpec((_B, 1), lambda cb: (0, 0)),
        pl.BlockSpec((_B, _CB), lambda cb: (0, cb)),
    ],
    out_specs=pl.BlockSpec(memory_space=pltpu.SMEM),
    out_shape=jax.ShapeDtypeStruct((1, 1), jnp.float32),
    scratch_shapes=[
        pltpu.VMEM((_B, 1), jnp.float32),
        pltpu.VMEM((_B, 1), jnp.float32),
    ],
    compiler_params=pltpu.CompilerParams(
        dimension_semantics=("arbitrary",),
    ),
)


def kernel(inp, label):
    tab = inp.reshape(_B * _V)
    g = _sc_gather()(tab, label)
    out = _tc_loss(label.reshape(_B, 1), g.reshape(_B, 1), inp)
    return out[0, 0]
